# baseline (speedup 1.0000x reference)
import jax
import jax.numpy as jnp
from jax import lax
from jax.experimental import pallas as pl
from jax.experimental.pallas import tpu as pltpu
from jax.experimental.pallas import tpu_sc as plsc

_B = 1024
_V = 100000
_CODE = 17
_NW = 16
_BLK = 128
_NBLK = _B // _BLK
_NUNIT = _CODE * _NBLK
_UPW = -(-_NUNIT // _NW)
_VROWS = 24
_NGRP = _BLK // 16

_LOG1P_COEFFS = (
    0.4054651,
    0.6666834,
    -0.22223203,
    0.0982361,
    -0.049072646,
    0.030434346,
    -0.017029611,
)


def _softplus(x):
    m = jnp.maximum(x, 0.0)
    t = jnp.exp(jnp.minimum(x, -x))
    w = t - 0.5
    acc = jnp.full((16,), _LOG1P_COEFFS[-1], dtype=jnp.float32)
    for c in reversed(_LOG1P_COEFFS[:-1]):
        acc = acc * w + c
    return m + acc


def _popcount(x):
    x = x - ((x >> 1) & 0x55555555)
    x = (x & 0x33333333) + ((x >> 2) & 0x33333333)
    x = (x + (x >> 4)) & 0x0F0F0F0F
    return (x * 0x01010101) >> 24


def _sc_body(scores_t_hbm, ci_hbm, out_hbm, ci_v, win_v, red_v, all_v, shr_v, sem):
    wid = lax.axis_index("s")

    ci_cp = pltpu.async_copy(ci_hbm, ci_v, sem)
    units = []
    copies = []
    for i in range(_UPW):
        u = wid + _NW * i
        level = u % _CODE
        blk = u // _CODE
        node_m1 = (jnp.int32(1) << level) - 1
        v0 = node_m1 & ~7
        if i == _UPW - 1:
            active = u < _NUNIT
            v0 = jnp.where(active, v0, 0)
            blk = jnp.where(active, blk, 0)
        units.append((u, level, node_m1 & 7, blk))
        v0 = pl.multiple_of(v0, 8)
        b0 = pl.multiple_of(blk * _BLK, _BLK)
        copies.append(
            pltpu.async_copy(
                scores_t_hbm.at[pl.ds(v0, _VROWS), pl.ds(b0, _BLK)],
                win_v.at[pl.ds(i * _VROWS, _VROWS), :],
                sem,
            )
        )
    ci_cp.wait()

    acc = jnp.zeros((16,), jnp.float32)
    for i, (u, level, off, blk) in enumerate(units):
        copies[i].wait()
        contrib = jnp.zeros((16,), jnp.float32)
        for g in range(_NGRP):
            c0 = pl.multiple_of(blk * _BLK + g * 16, 16)
            ci = ci_v[pl.ds(c0, 16)]
            bit = (ci >> (16 - level)) & 1
            cnt = _popcount(ci >> (_CODE - level))
            v_idx = i * _VROWS + off + cnt
            b_idx = g * 16 + lax.iota(jnp.int32, 16)
            v = plsc.load_gather(win_v, [v_idx, b_idx])
            x = jnp.where(bit != 0, v, -v)
            contrib = contrib + _softplus(x)
        if i == _UPW - 1:
            contrib = jnp.where(u < _NUNIT, contrib, 0.0)
        acc = acc + contrib

    red_v[...] = acc
    w0 = pl.multiple_of(wid * 16, 16)
    pltpu.sync_copy(red_v, shr_v.at[pl.ds(w0, 16)])
    plsc.subcore_barrier()

    @pl.when(wid == 0)
    def _():
        pltpu.sync_copy(shr_v, all_v)
        total = jnp.zeros((16,), jnp.float32)
        for w in range(_NW):
            total = total + all_v[pl.ds(w * 16, 16)]
        loss = jnp.sum(total) * (1.0 / _B)
        red_v[...] = jnp.full((16,), loss, dtype=jnp.float32)
        pltpu.sync_copy(red_v, out_hbm)


def _make_sc_loss():
    return pl.kernel(
        _sc_body,
        mesh=plsc.VectorSubcoreMesh(
            core_axis_name="c", subcore_axis_name="s", num_cores=1
        ),
        compiler_params=pltpu.CompilerParams(needs_layout_passes=False),
        out_type=jax.ShapeDtypeStruct((16,), jnp.float32),
        scratch_types=[
            pltpu.VMEM((_B,), jnp.int32),
            pltpu.VMEM((_UPW * _VROWS, _BLK), jnp.float32),
            pltpu.VMEM((16,), jnp.float32),
            pltpu.VMEM((_NW * 16,), jnp.float32),
            pltpu.VMEM_SHARED((_NW * 16,), jnp.float32),
            pltpu.SemaphoreType.DMA,
        ],
    )


def kernel(scores, class_indices):
    ci = class_indices.astype(jnp.int32)
    out = _make_sc_loss()(scores.T, ci)
    return out[0]


# speedup vs baseline: 1.0033x; 1.0033x over previous
import jax
import jax.numpy as jnp
from jax import lax
from jax.experimental import pallas as pl
from jax.experimental.pallas import tpu as pltpu
from jax.experimental.pallas import tpu_sc as plsc

_B = 1024
_V = 100000
_CODE = 17
_NW = 16
_BLK = 128
_NBLK = _B // _BLK
_NUNIT = _CODE * _NBLK
_UPW = -(-_NUNIT // _NW)
_VROWS = 24
_NGRP = _BLK // 16

_LOG1P_COEFFS = (
    0.4054651,
    0.6666834,
    -0.22223203,
    0.0982361,
    -0.049072646,
    0.030434346,
    -0.017029611,
)


def _softplus(x):
    m = jnp.maximum(x, 0.0)
    t = jnp.exp(jnp.minimum(x, -x))
    w = t - 0.5
    acc = jnp.full((16,), _LOG1P_COEFFS[-1], dtype=jnp.float32)
    for c in reversed(_LOG1P_COEFFS[:-1]):
        acc = acc * w + c
    return m + acc


def _popcount(x):
    x = x - ((x >> 1) & 0x55555555)
    x = (x & 0x33333333) + ((x >> 2) & 0x33333333)
    x = (x + (x >> 4)) & 0x0F0F0F0F
    return (x * 0x01010101) >> 24


def _sc_body(scores_t_hbm, ci_hbm, out_hbm, ci_v, win_v, red_v, all_v, shr_v, sem):
    wid = lax.axis_index("s")

    ci_cp = pltpu.async_copy(ci_hbm, ci_v, sem)
    units = []
    copies = []
    for i in range(_UPW):
        u = wid + _NW * i
        level = u % _CODE
        blk = u // _CODE
        node_m1 = (jnp.int32(1) << level) - 1
        v0 = node_m1 & ~7
        if i == _UPW - 1:
            active = u < _NUNIT
            v0 = jnp.where(active, v0, 0)
            blk = jnp.where(active, blk, 0)
        units.append((u, level, node_m1 & 7, blk))
        v0 = pl.multiple_of(v0, 8)
        b0 = pl.multiple_of(blk * _BLK, _BLK)
        copies.append(
            pltpu.async_copy(
                scores_t_hbm.at[pl.ds(v0, _VROWS), pl.ds(b0, _BLK)],
                win_v.at[pl.ds(i * _VROWS, _VROWS), :],
                sem,
            )
        )
    ci_cp.wait()

    acc = jnp.zeros((16,), jnp.float32)
    for i, (u, level, off, blk) in enumerate(units):
        copies[i].wait()
        contrib = jnp.zeros((16,), jnp.float32)
        for g in range(_NGRP):
            c0 = pl.multiple_of(blk * _BLK + g * 16, 16)
            ci = ci_v[pl.ds(c0, 16)]
            bit = (ci >> (16 - level)) & 1
            cnt = _popcount(ci >> (_CODE - level))
            v_idx = i * _VROWS + off + cnt
            b_idx = g * 16 + lax.iota(jnp.int32, 16)
            v = plsc.load_gather(win_v, [v_idx, b_idx])
            x = jnp.where(bit != 0, v, -v)
            contrib = contrib + _softplus(x)
        if i == _UPW - 1:
            contrib = jnp.where(u < _NUNIT, contrib, 0.0)
        acc = acc + contrib

    red_v[...] = acc
    w0 = pl.multiple_of(wid * 16, 16)
    pltpu.sync_copy(red_v, shr_v.at[pl.ds(w0, 16)])
    plsc.subcore_barrier()

    @pl.when(wid == 0)
    def _():
        pltpu.sync_copy(shr_v, all_v)
        total = jnp.zeros((16,), jnp.float32)
        for w in range(_NW):
            total = total + all_v[pl.ds(w * 16, 16)]
        loss = jnp.sum(total) * (1.0 / _B)
        red_v[...] = jnp.full((16,), loss, dtype=jnp.float32)
        pltpu.sync_copy(red_v, out_hbm)


def _make_sc_loss():
    return pl.kernel(
        _sc_body,
        mesh=plsc.VectorSubcoreMesh(
            core_axis_name="c", subcore_axis_name="s", num_cores=1
        ),
        compiler_params=pltpu.CompilerParams(
            needs_layout_passes=False, skip_device_barrier=True
        ),
        out_type=jax.ShapeDtypeStruct((16,), jnp.float32),
        scratch_types=[
            pltpu.VMEM((_B,), jnp.int32),
            pltpu.VMEM((_UPW * _VROWS, _BLK), jnp.float32),
            pltpu.VMEM((16,), jnp.float32),
            pltpu.VMEM((_NW * 16,), jnp.float32),
            pltpu.VMEM_SHARED((_NW * 16,), jnp.float32),
            pltpu.SemaphoreType.DMA,
        ],
    )


def kernel(scores, class_indices):
    ci = class_indices.astype(jnp.int32)
    out = _make_sc_loss()(scores.T, ci)
    return out[0]
